# Initial kernel scaffold; baseline (speedup 1.0000x reference)
import jax
import jax.numpy as jnp
from jax.experimental import pallas as pl


def kernel(message, attrs, W0, W1, W2):
    raise NotImplementedError("write your pallas kernel here")


# speedup vs baseline: 6.3299x; 6.3299x over previous
import functools

import jax
import jax.numpy as jnp
from jax import lax
from jax.experimental import pallas as pl
from jax.experimental.pallas import tpu as pltpu

E = 320000
D = 128
_B = 2000


def _tc_body(attrs_ref, msg_ref, w0_ref, w1_ref, w2_ref, out_ref):
    a0 = attrs_ref[:, 0:1]
    a1 = attrs_ref[:, 1:2]
    a2 = attrs_ref[:, 2:3]
    iota = lax.broadcasted_iota(jnp.int32, (attrs_ref.shape[0], 16), 1)
    oh = ((iota == a0) | (iota == a1 + 5) | (iota == a2 + 11)).astype(jnp.float32)
    wcat = jnp.concatenate(
        [w0_ref[:], w1_ref[:], w2_ref[:], jnp.zeros((3, D), jnp.float32)], axis=0
    )
    emb = jnp.dot(oh, wcat, preferred_element_type=jnp.float32)
    out_ref[:] = jnp.maximum(msg_ref[:] + emb, 0.0)


@jax.jit
def kernel(message, attrs, W0, W1, W2):
    attrs = attrs.astype(jnp.int32)
    grid = E // _B
    out = pl.pallas_call(
        _tc_body,
        grid=(grid,),
        in_specs=[
            pl.BlockSpec((_B, 3), lambda i: (i, 0)),
            pl.BlockSpec((_B, D), lambda i: (i, 0)),
            pl.BlockSpec((5, D), lambda i: (0, 0)),
            pl.BlockSpec((6, D), lambda i: (0, 0)),
            pl.BlockSpec((2, D), lambda i: (0, 0)),
        ],
        out_specs=pl.BlockSpec((_B, D), lambda i: (i, 0)),
        out_shape=jax.ShapeDtypeStruct((E, D), jnp.float32),
    )(attrs, message, W0, W1, W2)
    return out
